# baseline (speedup 1.0000x reference)
import functools

import jax
import jax.numpy as jnp
from jax import lax
from jax.experimental import pallas as pl
from jax.experimental.pallas import tpu as pltpu
from jax.experimental.pallas import tpu_sc as plsc

EMB = 64
NC = 2
NS = 16
NW = NC * NS
LANES = 16
UNROLL = 32


@functools.lru_cache(maxsize=None)
def _make_kernel(hist, batch, vocab):
    n_rounds = EMB // NW
    groups = batch // LANES
    mesh = plsc.VectorSubcoreMesh(core_axis_name="c", subcore_axis_name="s")

    scratch = [
        pltpu.VMEM((vocab,), jnp.float32),
        pltpu.VMEM((batch,), jnp.int32),
        pltpu.VMEM((batch,), jnp.int32),
        pltpu.VMEM((batch,), jnp.float32),
        pltpu.VMEM((batch,), jnp.float32),
        pltpu.SemaphoreType.DMA,
        pltpu.SemaphoreType.DMA,
        pltpu.SemaphoreType.DMA,
        pltpu.SemaphoreType.DMA,
    ]

    @functools.partial(
        pl.kernel,
        mesh=mesh,
        out_type=jax.ShapeDtypeStruct((hist, EMB, batch), jnp.float32),
        scratch_types=scratch,
        compiler_params=pltpu.CompilerParams(
            use_tc_tiling_on_sc=True, needs_layout_passes=False),
    )
    def k(idx_hbm, table_hbm, out_hbm, row_v, ix0, ix1, ov0, ov1,
          isem0, isem1, osem0, osem1):
        ix = (ix0, ix1)
        ov = (ov0, ov1)
        isem = (isem0, isem1)
        osem = (osem0, osem1)
        wid = lax.axis_index("s") * NC + lax.axis_index("c")

        def drain_out(b):
            pltpu.make_async_copy(ov[b], out_hbm.at[0, 0], osem[b]).wait()

        def gather_h(ixb, ovb):
            @plsc.parallel_loop(0, batch, LANES, unroll=UNROLL)
            def _(off):
                iv = ixb[pl.ds(off, LANES)]
                ovb[pl.ds(off, LANES)] = plsc.load_gather(row_v, [iv])

        for r in range(n_rounds):
            e = wid + NW * r
            pltpu.sync_copy(table_hbm.at[e], row_v)
            for b in range(2):
                pltpu.async_copy(idx_hbm.at[b], ix[b], isem[b])

            def h_pair(t, carry):
                for b in range(2):
                    h = t * 2 + b
                    pltpu.make_async_copy(
                        idx_hbm.at[h], ix[b], isem[b]).wait()
                    if r == 0:
                        @pl.when(h >= 2)
                        def _():
                            drain_out(b)
                    else:
                        drain_out(b)
                    gather_h(ix[b], ov[b])
                    pltpu.async_copy(ov[b], out_hbm.at[h, e], osem[b])

                    @pl.when(h + 2 < hist)
                    def _():
                        pltpu.async_copy(idx_hbm.at[h + 2], ix[b], isem[b])
                return carry

            lax.fori_loop(0, hist // 2, h_pair, 0)

        for b in range(2):
            drain_out(b)

    return k


def kernel(input_tensor, table):
    batch, hist = input_tensor.shape
    vocab, emb = table.shape
    idx_t = input_tensor.T.astype(jnp.int32)
    table_t = table.T
    out_t = _make_kernel(hist, batch, vocab)(idx_t, table_t)
    return jnp.transpose(out_t, (2, 0, 1))


# speedup vs baseline: 1.0030x; 1.0030x over previous
import functools

import jax
import jax.numpy as jnp
from jax import lax
from jax.experimental import pallas as pl
from jax.experimental.pallas import tpu as pltpu
from jax.experimental.pallas import tpu_sc as plsc

EMB = 64
NC = 2
NS = 16
NW = NC * NS
LANES = 16
UNROLL = 16


@functools.lru_cache(maxsize=None)
def _make_kernel(hist, batch, vocab):
    n_rounds = EMB // NW
    groups = batch // LANES
    mesh = plsc.VectorSubcoreMesh(core_axis_name="c", subcore_axis_name="s")

    scratch = [
        pltpu.VMEM((vocab,), jnp.float32),
        pltpu.VMEM((batch,), jnp.int32),
        pltpu.VMEM((batch,), jnp.int32),
        pltpu.VMEM((batch,), jnp.float32),
        pltpu.VMEM((batch,), jnp.float32),
        pltpu.SemaphoreType.DMA,
        pltpu.SemaphoreType.DMA,
        pltpu.SemaphoreType.DMA,
        pltpu.SemaphoreType.DMA,
    ]

    @functools.partial(
        pl.kernel,
        mesh=mesh,
        out_type=jax.ShapeDtypeStruct((hist, EMB, batch), jnp.float32),
        scratch_types=scratch,
        compiler_params=pltpu.CompilerParams(
            use_tc_tiling_on_sc=True, needs_layout_passes=False),
    )
    def k(idx_hbm, table_hbm, out_hbm, row_v, ix0, ix1, ov0, ov1,
          isem0, isem1, osem0, osem1):
        ix = (ix0, ix1)
        ov = (ov0, ov1)
        isem = (isem0, isem1)
        osem = (osem0, osem1)
        wid = lax.axis_index("s") * NC + lax.axis_index("c")

        def drain_out(b):
            pltpu.make_async_copy(ov[b], out_hbm.at[0, 0], osem[b]).wait()

        def gather_h(ixb, ovb):
            @plsc.parallel_loop(0, batch, LANES, unroll=UNROLL)
            def _(off):
                iv = ixb[pl.ds(off, LANES)]
                ovb[pl.ds(off, LANES)] = plsc.load_gather(row_v, [iv])

        for r in range(n_rounds):
            e = wid + NW * r
            pltpu.sync_copy(table_hbm.at[e], row_v)
            for b in range(2):
                pltpu.async_copy(idx_hbm.at[b], ix[b], isem[b])

            def h_pair(t, carry):
                for b in range(2):
                    h = t * 2 + b
                    pltpu.make_async_copy(
                        idx_hbm.at[h], ix[b], isem[b]).wait()
                    if r == 0:
                        @pl.when(h >= 2)
                        def _():
                            drain_out(b)
                    else:
                        drain_out(b)
                    gather_h(ix[b], ov[b])
                    pltpu.async_copy(ov[b], out_hbm.at[h, e], osem[b])

                    @pl.when(h + 2 < hist)
                    def _():
                        pltpu.async_copy(idx_hbm.at[h + 2], ix[b], isem[b])
                return carry

            lax.fori_loop(0, hist // 2, h_pair, 0)

        for b in range(2):
            drain_out(b)

    return k


def kernel(input_tensor, table):
    batch, hist = input_tensor.shape
    vocab, emb = table.shape
    idx_t = input_tensor.T.astype(jnp.int32)
    table_t = table.T
    out_t = _make_kernel(hist, batch, vocab)(idx_t, table_t)
    return jnp.transpose(out_t, (2, 0, 1))
